# baseline (speedup 1.0000x reference)
import jax
import jax.numpy as jnp
from jax import lax
from jax.experimental import pallas as pl
from jax.experimental.pallas import tpu as pltpu
from jax.experimental.pallas import tpu_sc as plsc

D = 1001
DP = 1024
DB = 896
DT = D - DB
NC, NS = 2, 16
NW = NC * NS
N = 4096 * 50
B_PER_W = N // NW
CH = 64
NCHUNK = B_PER_W // CH


def _exp_body(w_ref, o_ref):
    o_ref[...] = jnp.exp(w_ref[...])


_exp_table = pl.pallas_call(
    _exp_body,
    out_shape=jax.ShapeDtypeStruct((D, DP), jnp.float32),
)


def _gather_body(table_hbm, idx_hbm, out_hbm, idx_all, rows_v, tail_v, sem):
    wid = lax.axis_index("s") * NC + lax.axis_index("c")
    base = wid * B_PER_W
    pltpu.sync_copy(idx_hbm.at[pl.ds(base, B_PER_W)], idx_all)

    def chunk(c, carry):
        off = base + c * CH
        idx_c = idx_all.at[pl.ds(c * CH, CH)]
        pltpu.async_copy(table_hbm.at[idx_c], rows_v, sem).wait()
        pltpu.sync_copy(
            rows_v.at[:, pl.ds(0, DB)], out_hbm.at[pl.ds(off, CH), pl.ds(0, DB)]
        )

        @plsc.parallel_loop(0, CH, step=1, unroll=2)
        def _row(r):
            for j in range(6):
                tail_v[r, pl.ds(16 * j, 16)] = rows_v[r, pl.ds(DB + 16 * j, 16)]
            tail_v[r, pl.ds(DT - 16, 16)] = rows_v[r, pl.ds(D - 16, 16)]

        pltpu.sync_copy(tail_v, out_hbm.at[pl.ds(off, CH), pl.ds(DB, DT)])
        return carry

    lax.fori_loop(0, NCHUNK, chunk, 0)


_gather = pl.kernel(
    _gather_body,
    out_type=jax.ShapeDtypeStruct((N, D), jnp.float32),
    mesh=plsc.VectorSubcoreMesh(
        core_axis_name="c", subcore_axis_name="s", num_cores=NC, num_subcores=NS
    ),
    scratch_types=[
        pltpu.VMEM((B_PER_W,), jnp.int32),
        pltpu.VMEM((CH, DP), jnp.float32),
        pltpu.VMEM((CH, DT), jnp.float32),
        pltpu.SemaphoreType.DMA,
    ],
    compiler_params=pltpu.CompilerParams(use_tc_tiling_on_sc=True),
)


@jax.jit
def kernel(events, log_sigma_weight):
    w_pad = jnp.pad(log_sigma_weight, ((0, 0), (0, DP - D)))
    exp_table = _exp_table(w_pad)
    idx = events.reshape(N)
    out = _gather(exp_table, idx)
    return out.reshape(events.shape[0], events.shape[1], D)


# speedup vs baseline: 1.2427x; 1.2427x over previous
import jax
import jax.numpy as jnp
from jax import lax
from jax.experimental import pallas as pl
from jax.experimental.pallas import tpu as pltpu
from jax.experimental.pallas import tpu_sc as plsc

D = 1001
DP = 1024
DB = 896
DT = D - DB
NC, NS = 2, 16
NW = NC * NS
B = 4096
H = 50
B_PER_W = B // NW


def _exp_body(w_ref, o_ref):
    o_ref[...] = jnp.exp(w_ref[...])


_exp_table = pl.pallas_call(
    _exp_body,
    out_shape=jax.ShapeDtypeStruct((D, DP), jnp.float32),
)


def _gather_body(table_hbm, idx_hbm, out_hbm, idx_all, rows_v, tail_v, sem):
    wid = lax.axis_index("s") * NC + lax.axis_index("c")
    base = wid * B_PER_W
    pltpu.sync_copy(idx_hbm.at[pl.ds(base, B_PER_W), :], idx_all)

    def batch(bl, carry):
        b = base + bl
        pltpu.async_copy(table_hbm.at[idx_all.at[bl]], rows_v, sem).wait()
        pltpu.sync_copy(
            rows_v.at[:, pl.ds(0, DB)], out_hbm.at[b, :, pl.ds(0, DB)]
        )

        @plsc.parallel_loop(0, H, step=1, unroll=2)
        def _row(r):
            for j in range(6):
                tail_v[r, pl.ds(16 * j, 16)] = rows_v[r, pl.ds(DB + 16 * j, 16)]
            tail_v[r, pl.ds(DT - 16, 16)] = rows_v[r, pl.ds(D - 16, 16)]

        pltpu.sync_copy(tail_v, out_hbm.at[b, :, pl.ds(DB, DT)])
        return carry

    lax.fori_loop(0, B_PER_W, batch, 0)


_gather = pl.kernel(
    _gather_body,
    out_type=jax.ShapeDtypeStruct((B, H, D), jnp.float32),
    mesh=plsc.VectorSubcoreMesh(
        core_axis_name="c", subcore_axis_name="s", num_cores=NC, num_subcores=NS
    ),
    scratch_types=[
        pltpu.VMEM((B_PER_W, H), jnp.int32),
        pltpu.VMEM((H, DP), jnp.float32),
        pltpu.VMEM((H, DT), jnp.float32),
        pltpu.SemaphoreType.DMA,
    ],
    compiler_params=pltpu.CompilerParams(use_tc_tiling_on_sc=True),
)


@jax.jit
def kernel(events, log_sigma_weight):
    w_pad = jnp.pad(log_sigma_weight, ((0, 0), (0, DP - D)))
    exp_table = _exp_table(w_pad)
    return _gather(exp_table, events)


# speedup vs baseline: 3.4300x; 2.7601x over previous
import jax
import jax.numpy as jnp
from jax import lax
from jax.experimental import pallas as pl
from jax.experimental.pallas import tpu as pltpu
from jax.experimental.pallas import tpu_sc as plsc

D = 1001
DR = 1008
EP = 1024
NC, NS = 2, 16
NW = NC * NS
B = 4096
H = 50
NDT = 125
STRIPS = H * NDT
K_MAIN = -(-STRIPS // NW)
G = B // 16


def _exp_body(w_ref, o_ref):
    o_ref[...] = jnp.exp(w_ref[...])


_exp_tableT = pl.pallas_call(
    _exp_body,
    out_shape=jax.ShapeDtypeStruct((DR, EP), jnp.float32),
)


def _gather_body(expT_hbm, evT_hbm, out_hbm, idx1, rows2, strip2, last1,
                 lsem, ssem):
    wid = lax.axis_index("s") * NC + lax.axis_index("c")
    dsp = [jnp.full((16,), d, jnp.int32) for d in range(8)]

    def strip_of(k):
        s = wid + NW * k
        return s // NDT, s % NDT, s < STRIPS

    def issue_loads(k):
        h, dt, ok = strip_of(k)
        b = k % 2

        @pl.when(ok)
        def _():
            pltpu.async_copy(evT_hbm.at[pl.ds(h * B, B)], idx1.at[pl.ds(b * B, B)],
                             lsem.at[b])
            pltpu.async_copy(expT_hbm.at[pl.ds(dt * 8, 8), :], rows2.at[b],
                             lsem.at[b])

    issue_loads(0)

    def step(k, carry):
        h, dt, ok = strip_of(k)
        b = k % 2

        @pl.when(ok)
        def _wait_loads():
            pltpu.make_async_copy(
                evT_hbm.at[pl.ds(0, B)], idx1.at[pl.ds(b * B, B)], lsem.at[b]
            ).wait()
            pltpu.make_async_copy(
                expT_hbm.at[pl.ds(0, 8), :], rows2.at[b], lsem.at[b]
            ).wait()

        issue_loads(k + 1)

        @pl.when(ok)
        def _compute_store():
            @pl.when(k >= 2)
            def _drain_prev():
                pltpu.make_async_copy(
                    strip2.at[b], out_hbm.at[0, pl.ds(0, 8), :], ssem.at[b]
                ).wait()

            bB = b * B

            @plsc.parallel_loop(0, G, step=1, unroll=2)
            def _g(g):
                e16 = idx1[pl.ds(bB + g * 16, 16)]
                for d in range(8):
                    strip2[b, d, pl.ds(g * 16, 16)] = plsc.load_gather(
                        rows2.at[b], [dsp[d], e16]
                    )

            pltpu.async_copy(
                strip2.at[b], out_hbm.at[h, pl.ds(dt * 8, 8), :], ssem.at[b]
            )

        return carry

    lax.fori_loop(0, K_MAIN, step, 0)

    for j in (K_MAIN - 2, K_MAIN - 1):
        _, _, ok = strip_of(j)

        @pl.when(ok)
        def _drain_tail(j=j):
            pltpu.make_async_copy(
                strip2.at[j % 2], out_hbm.at[0, pl.ds(0, 8), :], ssem.at[j % 2]
            ).wait()

    def last_d(t, carry):
        h = wid + NW * t

        @pl.when(h < H)
        def _():
            pltpu.sync_copy(evT_hbm.at[pl.ds(h * B, B)], idx1.at[pl.ds(0, B)])
            pltpu.sync_copy(expT_hbm.at[pl.ds(NDT * 8, 8), :], rows2.at[0])

            @plsc.parallel_loop(0, G, step=1, unroll=2)
            def _g(g):
                e16 = idx1[pl.ds(g * 16, 16)]
                last1[0, pl.ds(g * 16, 16)] = plsc.load_gather(
                    rows2.at[0], [dsp[0], e16]
                )

            pltpu.sync_copy(last1, out_hbm.at[h, pl.ds(NDT * 8, 1), :])

        return carry

    lax.fori_loop(0, 2, last_d, 0)


_gather = pl.kernel(
    _gather_body,
    out_type=jax.ShapeDtypeStruct((H, D, B), jnp.float32),
    mesh=plsc.VectorSubcoreMesh(
        core_axis_name="c", subcore_axis_name="s", num_cores=NC, num_subcores=NS
    ),
    scratch_types=[
        pltpu.VMEM((2 * B,), jnp.int32),
        pltpu.VMEM((2, 8, EP), jnp.float32),
        pltpu.VMEM((2, 8, B), jnp.float32),
        pltpu.VMEM((1, B), jnp.float32),
        pltpu.SemaphoreType.DMA((2,)),
        pltpu.SemaphoreType.DMA((2,)),
    ],
    compiler_params=pltpu.CompilerParams(
        use_tc_tiling_on_sc=True, needs_layout_passes=False
    ),
)


@jax.jit
def kernel(events, log_sigma_weight):
    w_tp = jnp.pad(log_sigma_weight.T, ((0, DR - D), (0, EP - D)))
    expT = _exp_tableT(w_tp)
    evT = events.T.reshape(H * B)
    ot = _gather(expT, evT)
    return jnp.transpose(ot, (2, 0, 1))


# speedup vs baseline: 3.4328x; 1.0008x over previous
import jax
import jax.numpy as jnp
from jax import lax
from jax.experimental import pallas as pl
from jax.experimental.pallas import tpu as pltpu
from jax.experimental.pallas import tpu_sc as plsc

D = 1001
DR = 1008
EP = 1024
NC, NS = 2, 16
NW = NC * NS
B = 4096
H = 50
NDT = 125
STRIPS = H * NDT
K_MAIN = -(-STRIPS // NW)
G = B // 16


def _exp_body(w_ref, o_ref):
    o_ref[...] = jnp.exp(w_ref[...])


_exp_tableT = pl.pallas_call(
    _exp_body,
    out_shape=jax.ShapeDtypeStruct((DR, EP), jnp.float32),
)


def _gather_body(expT_hbm, evT_hbm, out_hbm, idx1, rows2, strip2, last1,
                 lsem, ssem):
    wid = lax.axis_index("s") * NC + lax.axis_index("c")
    dsp = [jnp.full((16,), d, jnp.int32) for d in range(8)]

    def strip_of(k):
        s = wid + NW * k
        return s // NDT, s % NDT, s < STRIPS

    def issue_loads(k):
        h, dt, ok = strip_of(k)
        b = k % 2

        @pl.when(ok)
        def _():
            pltpu.async_copy(evT_hbm.at[pl.ds(h * B, B)], idx1.at[pl.ds(b * B, B)],
                             lsem.at[b])
            pltpu.async_copy(expT_hbm.at[pl.ds(dt * 8, 8), :], rows2.at[b],
                             lsem.at[b])

    issue_loads(0)

    def step(k, carry):
        h, dt, ok = strip_of(k)
        b = k % 2

        @pl.when(ok)
        def _wait_loads():
            pltpu.make_async_copy(
                evT_hbm.at[pl.ds(0, B)], idx1.at[pl.ds(b * B, B)], lsem.at[b]
            ).wait()
            pltpu.make_async_copy(
                expT_hbm.at[pl.ds(0, 8), :], rows2.at[b], lsem.at[b]
            ).wait()

        issue_loads(k + 1)

        @pl.when(ok)
        def _compute_store():
            @pl.when(k >= 2)
            def _drain_prev():
                pltpu.make_async_copy(
                    strip2.at[b], out_hbm.at[0, pl.ds(0, 8), :], ssem.at[b]
                ).wait()

            bB = b * B

            @plsc.parallel_loop(0, G, step=1, unroll=4)
            def _g(g):
                e16 = idx1[pl.ds(bB + g * 16, 16)]
                for d in range(8):
                    strip2[b, d, pl.ds(g * 16, 16)] = plsc.load_gather(
                        rows2.at[b], [dsp[d], e16]
                    )

            pltpu.async_copy(
                strip2.at[b], out_hbm.at[h, pl.ds(dt * 8, 8), :], ssem.at[b]
            )

        return carry

    lax.fori_loop(0, K_MAIN, step, 0)

    for j in (K_MAIN - 2, K_MAIN - 1):
        _, _, ok = strip_of(j)

        @pl.when(ok)
        def _drain_tail(j=j):
            pltpu.make_async_copy(
                strip2.at[j % 2], out_hbm.at[0, pl.ds(0, 8), :], ssem.at[j % 2]
            ).wait()

    def last_d(t, carry):
        h = wid + NW * t

        @pl.when(h < H)
        def _():
            pltpu.sync_copy(evT_hbm.at[pl.ds(h * B, B)], idx1.at[pl.ds(0, B)])
            pltpu.sync_copy(expT_hbm.at[pl.ds(NDT * 8, 8), :], rows2.at[0])

            @plsc.parallel_loop(0, G, step=1, unroll=2)
            def _g(g):
                e16 = idx1[pl.ds(g * 16, 16)]
                last1[0, pl.ds(g * 16, 16)] = plsc.load_gather(
                    rows2.at[0], [dsp[0], e16]
                )

            pltpu.sync_copy(last1, out_hbm.at[h, pl.ds(NDT * 8, 1), :])

        return carry

    lax.fori_loop(0, 2, last_d, 0)


_gather = pl.kernel(
    _gather_body,
    out_type=jax.ShapeDtypeStruct((H, D, B), jnp.float32),
    mesh=plsc.VectorSubcoreMesh(
        core_axis_name="c", subcore_axis_name="s", num_cores=NC, num_subcores=NS
    ),
    scratch_types=[
        pltpu.VMEM((2 * B,), jnp.int32),
        pltpu.VMEM((2, 8, EP), jnp.float32),
        pltpu.VMEM((2, 8, B), jnp.float32),
        pltpu.VMEM((1, B), jnp.float32),
        pltpu.SemaphoreType.DMA((2,)),
        pltpu.SemaphoreType.DMA((2,)),
    ],
    compiler_params=pltpu.CompilerParams(
        use_tc_tiling_on_sc=True, needs_layout_passes=False
    ),
)


@jax.jit
def kernel(events, log_sigma_weight):
    w_tp = jnp.pad(log_sigma_weight.T, ((0, DR - D), (0, EP - D)))
    expT = _exp_tableT(w_tp)
    evT = events.T.reshape(H * B)
    ot = _gather(expT, evT)
    return jnp.transpose(ot, (2, 0, 1))


# speedup vs baseline: 3.4521x; 1.0056x over previous
import jax
import jax.numpy as jnp
from jax import lax
from jax.experimental import pallas as pl
from jax.experimental.pallas import tpu as pltpu
from jax.experimental.pallas import tpu_sc as plsc

D = 1001
DR = 1008
EP = 1024
NC, NS = 2, 16
NW = NC * NS
B = 4096
H = 50
NDT = 125
STRIPS = H * NDT
K_MAIN = -(-STRIPS // NW)
G = B // 16


def _exp_body(w_ref, o_ref):
    o_ref[...] = jnp.exp(w_ref[...])


_exp_tableT = pl.pallas_call(
    _exp_body,
    out_shape=jax.ShapeDtypeStruct((DR, EP), jnp.float32),
)


def _gather_body(expT_hbm, evT_hbm, out_hbm, idx1, rows2, strip2, last1,
                 lsem, ssem):
    wid = lax.axis_index("s") * NC + lax.axis_index("c")
    dsp = [jnp.full((16,), d, jnp.int32) for d in range(8)]

    def strip_of(k):
        s = wid + NW * k
        return s // NDT, s % NDT, s < STRIPS

    def issue_loads(k):
        h, dt, ok = strip_of(k)
        b = k % 2

        @pl.when(ok)
        def _():
            pltpu.async_copy(evT_hbm.at[pl.ds(h * B, B)], idx1.at[pl.ds(b * B, B)],
                             lsem.at[b])
            pltpu.async_copy(expT_hbm.at[pl.ds(dt * 8, 8), :], rows2.at[b],
                             lsem.at[b])

    issue_loads(0)

    def step(k, carry):
        h, dt, ok = strip_of(k)
        b = k % 2

        @pl.when(ok)
        def _wait_loads():
            pltpu.make_async_copy(
                evT_hbm.at[pl.ds(0, B)], idx1.at[pl.ds(b * B, B)], lsem.at[b]
            ).wait()
            pltpu.make_async_copy(
                expT_hbm.at[pl.ds(0, 8), :], rows2.at[b], lsem.at[b]
            ).wait()

        issue_loads(k + 1)

        @pl.when(ok)
        def _compute_store():
            @pl.when(k >= 2)
            def _drain_prev():
                pltpu.make_async_copy(
                    strip2.at[b], out_hbm.at[0, pl.ds(0, 8), :], ssem.at[b]
                ).wait()

            bB = b * B

            @plsc.parallel_loop(0, G, step=1, unroll=2)
            def _g(g):
                e16 = idx1[pl.ds(bB + g * 16, 16)]
                for d in range(8):
                    strip2[b, d, pl.ds(g * 16, 16)] = plsc.load_gather(
                        rows2.at[b], [dsp[d], e16]
                    )

            pltpu.async_copy(
                strip2.at[b], out_hbm.at[h, pl.ds(dt * 8, 8), :], ssem.at[b]
            )

        return carry

    lax.fori_loop(0, K_MAIN, step, 0)

    for j in (K_MAIN - 2, K_MAIN - 1):
        _, _, ok = strip_of(j)

        @pl.when(ok)
        def _drain_tail(j=j):
            pltpu.make_async_copy(
                strip2.at[j % 2], out_hbm.at[0, pl.ds(0, 8), :], ssem.at[j % 2]
            ).wait()

    def last_d(t, carry):
        h = wid + NW * t

        @pl.when(h < H)
        def _():
            pltpu.sync_copy(evT_hbm.at[pl.ds(h * B, B)], idx1.at[pl.ds(0, B)])
            pltpu.sync_copy(expT_hbm.at[pl.ds(NDT * 8, 8), :], rows2.at[0])

            @plsc.parallel_loop(0, G, step=1, unroll=2)
            def _g(g):
                e16 = idx1[pl.ds(g * 16, 16)]
                last1[0, pl.ds(g * 16, 16)] = plsc.load_gather(
                    rows2.at[0], [dsp[0], e16]
                )

            pltpu.sync_copy(last1, out_hbm.at[h, pl.ds(NDT * 8, 1), :])

        return carry

    lax.fori_loop(0, 2, last_d, 0)


_gather = pl.kernel(
    _gather_body,
    out_type=jax.ShapeDtypeStruct((H, D, B), jnp.float32),
    mesh=plsc.VectorSubcoreMesh(
        core_axis_name="c", subcore_axis_name="s", num_cores=NC, num_subcores=NS
    ),
    scratch_types=[
        pltpu.VMEM((2 * B,), jnp.int32),
        pltpu.VMEM((2, 8, EP), jnp.float32),
        pltpu.VMEM((2, 8, B), jnp.float32),
        pltpu.VMEM((1, B), jnp.float32),
        pltpu.SemaphoreType.DMA((2,)),
        pltpu.SemaphoreType.DMA((2,)),
    ],
    compiler_params=pltpu.CompilerParams(
        use_tc_tiling_on_sc=True, needs_layout_passes=False
    ),
)


@jax.jit
def kernel(events, log_sigma_weight):
    w_tp = jnp.pad(log_sigma_weight.T, ((0, DR - D), (0, EP - D)))
    expT = _exp_tableT(w_tp)
    evT = events.T.reshape(H * B)
    ot = _gather(expT, evT)
    return jnp.transpose(ot, (2, 0, 1))
